# Initial kernel scaffold; baseline (speedup 1.0000x reference)
import jax
import jax.numpy as jnp
from jax.experimental import pallas as pl


def kernel(features, points, normals, ranges, win1, bin1, win2, bin2, gni_w, gni_b, a1, cb1, a2, cb2, wo1, bo1, wo2, bo2, gno_w, gno_b, ll1_w, ll1_b, ll2_w, ll2_b, lt_w, lt_b):
    raise NotImplementedError("write your pallas kernel here")


# speedup vs baseline: 1.7647x; 1.7647x over previous
import functools
import math

import jax
import jax.numpy as jnp
from jax.experimental import pallas as pl
from jax.experimental.pallas import tpu as pltpu

N = 2048
H = 128
C = 8
GROUPS = 4
EPS = 1e-5
RADIUS = 9.0

BI = 256
BJ = 512
NI = N // BI
NJ = N // BJ


def _leaky(x):
    return jnp.where(x >= 0, x, 0.2 * x)


def _group_norm(x, gamma, beta):
    gs = H // GROUPS
    cols = []
    for g in range(GROUPS):
        sub = x[:, g * gs:(g + 1) * gs]
        m = jnp.mean(sub)
        v = jnp.mean((sub - m) * (sub - m))
        cols.append((sub - m) * jax.lax.rsqrt(v + EPS))
    y = jnp.concatenate(cols, axis=1)
    return y * gamma + beta


def _mlp_in_body(x_ref, w1_ref, b1_ref, w2_ref, b2_ref, gw_ref, gb_ref, o_ref):
    f = jnp.dot(x_ref[...], w1_ref[...], preferred_element_type=jnp.float32)
    f = _leaky(f + b1_ref[...])
    f = jnp.dot(f, w2_ref[...], preferred_element_type=jnp.float32)
    f = _leaky(f + b2_ref[...])
    o_ref[...] = _group_norm(f, gw_ref[...], gb_ref[...])


def _mlp_out_body(u_ref, x_ref, wo1_ref, bo1_ref, wo2_ref, bo2_ref,
                  gw_ref, gb_ref, l1_ref, l1b_ref, l2_ref, l2b_ref,
                  lt_ref, ltb_ref, o_ref):
    o = jnp.dot(u_ref[...], wo1_ref[...], preferred_element_type=jnp.float32)
    o = _leaky(o + bo1_ref[...])
    o = jnp.dot(o, wo2_ref[...], preferred_element_type=jnp.float32)
    o = _leaky(o + bo2_ref[...])
    o = _group_norm(o, gw_ref[...], gb_ref[...])
    xi = jnp.dot(o, l1_ref[...], preferred_element_type=jnp.float32)
    xi = jnp.maximum(xi + l1b_ref[...], 0.0)
    xi = jnp.dot(xi, l2_ref[...], preferred_element_type=jnp.float32) + l2b_ref[...]
    xn = jnp.dot(x_ref[...], lt_ref[...], preferred_element_type=jnp.float32)
    o_ref[...] = xn + ltb_ref[...] + xi


def _pair_body(ip_ref, jp_ref, f_ref, a2t_ref, cb2_ref, o_ref):
    j = pl.program_id(1)
    ip = ip_ref[...]
    jp = jp_ref[...]
    fj = f_ref[...]

    pxj = jp[0:1, :]
    pyj = jp[1:2, :]
    pzj = jp[2:3, :]
    pn2j = jp[3:4, :]
    nxj = jp[4:5, :]
    nyj = jp[5:6, :]
    nzj = jp[6:7, :]

    pxi = ip[:, 0:1]
    pyi = ip[:, 1:2]
    pzi = ip[:, 2:3]
    pn2i = ip[:, 3:4]
    nxi = ip[:, 4:5]
    nyi = ip[:, 5:6]
    nzi = ip[:, 6:7]

    pij = pxi * pxj + pyi * pyj + pzi * pzj
    sq = pn2j + pn2i - 2.0 * pij
    dot = nxi * nxj + nyi * nyj + nzi * nzj
    e = 2.0 - dot
    w = jnp.exp(-sq * (e * e))

    acc = jnp.dot(w, fj, preferred_element_type=jnp.float32) * cb2_ref[...]
    for c in range(C):
        a0 = ip[:, 8 + 3 * c:9 + 3 * c]
        a1c = ip[:, 9 + 3 * c:10 + 3 * c]
        a2c = ip[:, 10 + 3 * c:11 + 3 * c]
        bias = ip[:, 32 + c:33 + c]
        t = a0 * pxj + a1c * pyj + a2c * pzj + bias
        r = jnp.maximum(t, 0.0) * w
        acc += (jnp.dot(r, fj, preferred_element_type=jnp.float32)
                * a2t_ref[c:c + 1, :])

    @pl.when(j == 0)
    def _():
        o_ref[...] = acc

    @pl.when(j != 0)
    def _():
        o_ref[...] += acc


_mlp_in_call = pl.pallas_call(
    _mlp_in_body,
    out_shape=jax.ShapeDtypeStruct((N, H), jnp.float32),
)

_mlp_out_call = pl.pallas_call(
    _mlp_out_body,
    out_shape=jax.ShapeDtypeStruct((N, H), jnp.float32),
)

_pair_call = pl.pallas_call(
    _pair_body,
    grid=(NI, NJ),
    in_specs=[
        pl.BlockSpec((BI, 40), lambda i, j: (i, 0)),
        pl.BlockSpec((8, BJ), lambda i, j: (0, j)),
        pl.BlockSpec((BJ, H), lambda i, j: (j, 0)),
        pl.BlockSpec((C, H), lambda i, j: (0, 0)),
        pl.BlockSpec((1, H), lambda i, j: (0, 0)),
    ],
    out_specs=pl.BlockSpec((BI, H), lambda i, j: (i, 0)),
    out_shape=jax.ShapeDtypeStruct((N, H), jnp.float32),
    compiler_params=pltpu.CompilerParams(
        dimension_semantics=("parallel", "arbitrary"),
    ),
)


def kernel(features, points, normals, ranges, win1, bin1, win2, bin2, gni_w,
           gni_b, a1, cb1, a2, cb2, wo1, bo1, wo2, bo2, gno_w, gno_b, ll1_w,
           ll1_b, ll2_w, ll2_b, lt_w, lt_b):
    p = points * (1.0 / (math.sqrt(2.0) * RADIUS))
    pn2 = jnp.sum(p * p, axis=1, keepdims=True)
    nrm = normals[:, 0, :]
    zcol = jnp.zeros((N, 1), jnp.float32)

    jpack = jnp.concatenate([p, pn2, nrm, zcol], axis=1).T

    x = features
    L = win1.shape[0]
    for l in range(L):
        f = _mlp_in_call(
            x, win1[l].T, bin1[l][None, :], win2[l].T, bin2[l][None, :],
            gni_w[l][None, :], gni_b[l][None, :])

        A = jnp.einsum("ck,ikm->icm", a1[l], normals)
        bias = cb1[l][None, :] - jnp.einsum("icm,im->ic", A, p)
        ipack = jnp.concatenate(
            [p, pn2, nrm, zcol, A.reshape(N, 3 * C), bias], axis=1)

        u = _pair_call(ipack, jpack, f, a2[l].T, cb2[l][None, :])

        x = _mlp_out_call(
            u, x, wo1[l].T, bo1[l][None, :], wo2[l].T, bo2[l][None, :],
            gno_w[l][None, :], gno_b[l][None, :], ll1_w[l].T,
            ll1_b[l][None, :], ll2_w[l].T, ll2_b[l][None, :], lt_w[l].T,
            lt_b[l][None, :])
    return x


# speedup vs baseline: 2.1507x; 1.2188x over previous
import functools
import math

import jax
import jax.numpy as jnp
from jax.experimental import pallas as pl
from jax.experimental.pallas import tpu as pltpu

N = 2048
H = 128
C = 8
GROUPS = 4
EPS = 1e-5
RADIUS = 9.0

BI = 256
BJ = 512
NI = N // BI
NJ = N // BJ


def _leaky(x):
    return jnp.where(x >= 0, x, 0.2 * x)


def _group_norm(x, gamma, beta):
    gs = H // GROUPS
    cols = []
    for g in range(GROUPS):
        sub = x[:, g * gs:(g + 1) * gs]
        m = jnp.mean(sub)
        v = jnp.mean((sub - m) * (sub - m))
        cols.append((sub - m) * jax.lax.rsqrt(v + EPS))
    y = jnp.concatenate(cols, axis=1)
    return y * gamma + beta


def _mlp_in_body(x_ref, w1_ref, b1_ref, w2_ref, b2_ref, gw_ref, gb_ref, o_ref):
    f = jnp.dot(x_ref[...], w1_ref[...], preferred_element_type=jnp.float32)
    f = _leaky(f + b1_ref[...])
    f = jnp.dot(f, w2_ref[...], preferred_element_type=jnp.float32)
    f = _leaky(f + b2_ref[...])
    o_ref[...] = _group_norm(f, gw_ref[...], gb_ref[...])


def _mlp_out_body(u_ref, x_ref, wo1_ref, bo1_ref, wo2_ref, bo2_ref,
                  gw_ref, gb_ref, l1_ref, l1b_ref, l2_ref, l2b_ref,
                  lt_ref, ltb_ref, o_ref):
    o = jnp.dot(u_ref[...], wo1_ref[...], preferred_element_type=jnp.float32)
    o = _leaky(o + bo1_ref[...])
    o = jnp.dot(o, wo2_ref[...], preferred_element_type=jnp.float32)
    o = _leaky(o + bo2_ref[...])
    o = _group_norm(o, gw_ref[...], gb_ref[...])
    xi = jnp.dot(o, l1_ref[...], preferred_element_type=jnp.float32)
    xi = jnp.maximum(xi + l1b_ref[...], 0.0)
    xi = jnp.dot(xi, l2_ref[...], preferred_element_type=jnp.float32) + l2b_ref[...]
    xn = jnp.dot(x_ref[...], lt_ref[...], preferred_element_type=jnp.float32)
    o_ref[...] = xn + ltb_ref[...] + xi


def _pair_body(u_ref, jp_ref, f_ref, a2t_ref, cb2_ref, o_ref):
    j = pl.program_id(1)
    jp = jp_ref[...]
    fj = f_ref[...]

    um = u_ref[...].reshape(NG * BI, 8)
    tt = jnp.dot(um, jp, preferred_element_type=jnp.float32)

    sq = tt[0:BI]
    e = tt[BI:2 * BI]
    w = jnp.exp(-sq * (e * e))

    acc = jnp.dot(w, fj, preferred_element_type=jnp.float32) * cb2_ref[...]
    for c in range(C):
        r = jnp.maximum(tt[(2 + c) * BI:(3 + c) * BI], 0.0) * w
        acc += (jnp.dot(r, fj, preferred_element_type=jnp.float32)
                * a2t_ref[c:c + 1, :])

    @pl.when(j == 0)
    def _():
        o_ref[...] = acc

    @pl.when(j != 0)
    def _():
        o_ref[...] += acc


_mlp_in_call = pl.pallas_call(
    _mlp_in_body,
    out_shape=jax.ShapeDtypeStruct((N, H), jnp.float32),
)

_mlp_out_call = pl.pallas_call(
    _mlp_out_body,
    out_shape=jax.ShapeDtypeStruct((N, H), jnp.float32),
)

NG = 2 + C

_pair_call = pl.pallas_call(
    _pair_body,
    grid=(NI, NJ),
    in_specs=[
        pl.BlockSpec((NG, BI, 8), lambda i, j: (0, i, 0)),
        pl.BlockSpec((8, BJ), lambda i, j: (0, j)),
        pl.BlockSpec((BJ, H), lambda i, j: (j, 0)),
        pl.BlockSpec((C, H), lambda i, j: (0, 0)),
        pl.BlockSpec((1, H), lambda i, j: (0, 0)),
    ],
    out_specs=pl.BlockSpec((BI, H), lambda i, j: (i, 0)),
    out_shape=jax.ShapeDtypeStruct((N, H), jnp.float32),
    compiler_params=pltpu.CompilerParams(
        dimension_semantics=("parallel", "arbitrary"),
    ),
)


def kernel(features, points, normals, ranges, win1, bin1, win2, bin2, gni_w,
           gni_b, a1, cb1, a2, cb2, wo1, bo1, wo2, bo2, gno_w, gno_b, ll1_w,
           ll1_b, ll2_w, ll2_b, lt_w, lt_b):
    p = points * (1.0 / (math.sqrt(2.0) * RADIUS))
    pn2 = jnp.sum(p * p, axis=1, keepdims=True)
    nrm = normals[:, 0, :]
    zcol = jnp.zeros((N, 1), jnp.float32)
    ocol = jnp.ones((N, 1), jnp.float32)

    jpack = jnp.concatenate([p, pn2, nrm, ocol], axis=1).T

    u_sq = jnp.concatenate([-2.0 * p, ocol, jnp.zeros((N, 3), jnp.float32),
                            pn2], axis=1)
    u_e = jnp.concatenate([jnp.zeros((N, 4), jnp.float32), -nrm,
                           2.0 * ocol], axis=1)

    x = features
    L = win1.shape[0]
    for l in range(L):
        f = _mlp_in_call(
            x, win1[l].T, bin1[l][None, :], win2[l].T, bin2[l][None, :],
            gni_w[l][None, :], gni_b[l][None, :])

        A = jnp.einsum("ck,ikm->icm", a1[l], normals)
        bias = cb1[l][None, :] - jnp.einsum("icm,im->ic", A, p)
        u_t = jnp.concatenate(
            [A, jnp.zeros((N, C, 4), jnp.float32), bias[:, :, None]],
            axis=2)
        ubig = jnp.concatenate(
            [u_sq[None], u_e[None], jnp.moveaxis(u_t, 1, 0)], axis=0)

        u = _pair_call(ubig, jpack, f, a2[l].T, cb2[l][None, :])

        x = _mlp_out_call(
            u, x, wo1[l].T, bo1[l][None, :], wo2[l].T, bo2[l][None, :],
            gno_w[l][None, :], gno_b[l][None, :], ll1_w[l].T,
            ll1_b[l][None, :], ll2_w[l].T, ll2_b[l][None, :], lt_w[l].T,
            lt_b[l][None, :])
    return x
